# Initial kernel scaffold; baseline (speedup 1.0000x reference)
import jax
import jax.numpy as jnp
from jax.experimental import pallas as pl


def kernel(x, edge_index, W, b):
    raise NotImplementedError("write your pallas kernel here")


# speedup vs baseline: 7.1899x; 7.1899x over previous
import functools

import jax
import jax.numpy as jnp
from jax import lax
from jax.experimental import pallas as pl
from jax.experimental.pallas import tpu as pltpu
from jax.experimental.pallas import tpu_sc as plsc

N_NODES = 10000
D = 128
E = 320000
NC, NS = 2, 16
HALF = 5120
ACC_R = HALF + 8
RPS = HALF // NS
E_W = E // NS
K = 80
STEPS = E_W // K
NH = (N_NODES + 127) // 128 + 1


def _sc_aggregate(x, src, dst):
    mesh = plsc.VectorSubcoreMesh(core_axis_name="c", subcore_axis_name="s")

    @functools.partial(
        pl.kernel,
        mesh=mesh,
        out_type=jax.ShapeDtypeStruct((NC * HALF, D), jnp.float32),
        scratch_types=[
            pltpu.VMEM((K,), jnp.int32),
            pltpu.VMEM((K,), jnp.int32),
            pltpu.VMEM((K, D), jnp.float32),
            pltpu.VMEM((K,), jnp.int32),
            pltpu.VMEM((K,), jnp.int32),
            pltpu.VMEM((K, D), jnp.float32),
            pltpu.VMEM((E_W,), jnp.int32),
            pltpu.VMEM((E_W,), jnp.int32),
            pltpu.VMEM_SHARED((ACC_R, D), jnp.float32),
            pltpu.SemaphoreType.DMA,
            pltpu.SemaphoreType.DMA,
        ],
    )
    def agg_kernel(x_hbm, src_hbm, dst_hbm, agg_out,
                   src_v, dst_v, rows_v, src_b, dst_b, rows_b,
                   srcall_v, dstall_v, acc_sh, sem_a, sem_b):
        c = lax.axis_index("c")
        s = lax.axis_index("s")

        zero16 = jnp.zeros((16,), jnp.float32)

        def zero_rows(i, _):
            r = i // (D // 16)
            j = i % (D // 16)
            rows_v[r, pl.ds(j * 16, 16)] = zero16
            return 0
        lax.fori_loop(0, K * (D // 16), zero_rows, 0)

        base_r = s * RPS
        for j in range(RPS // K):
            pltpu.sync_copy(rows_v, acc_sh.at[pl.ds(base_r + j * K, K)])
        pltpu.sync_copy(rows_v.at[pl.ds(0, 8)], acc_sh.at[pl.ds(HALF, 8)])

        plsc.subcore_barrier()

        lo = c * HALF

        pltpu.sync_copy(src_hbm.at[pl.ds(s * E_W, E_W)], srcall_v)
        pltpu.sync_copy(dst_hbm.at[pl.ds(s * E_W, E_W)], dstall_v)

        def rm_all(j, _):
            d16 = dstall_v[pl.ds(j * 16, 16)] - lo
            keep = (d16 >= 0) & (d16 < HALF)
            dstall_v[pl.ds(j * 16, 16)] = jnp.where(keep, d16, HALF)
            return 0
        lax.fori_loop(0, E_W // 16, rm_all, 0)

        def load_src(buf, cidx):
            for t in range(K // 16):
                buf[pl.ds(t * 16, 16)] = srcall_v[pl.ds(cidx * K + t * 16, 16)]

        def load_dst(buf, cidx):
            for t in range(K // 16):
                buf[pl.ds(t * 16, 16)] = dstall_v[pl.ds(cidx * K + t * 16, 16)]

        load_src(src_v, 0)
        pltpu.async_copy(x_hbm.at[src_v], rows_v, sem_a)
        load_dst(dst_v, 0)

        def pair(g, _):
            c1 = 2 * g + 1
            load_src(src_b, c1)
            pltpu.async_copy(x_hbm.at[src_b], rows_b, sem_b)
            load_dst(dst_b, c1)
            pltpu.make_async_copy(x_hbm.at[src_v], rows_v, sem_a).wait()
            pltpu.sync_copy(rows_v, acc_sh.at[dst_v], add=True)
            load_src(src_v, c1 + 1)
            pltpu.async_copy(x_hbm.at[src_v], rows_v, sem_a)
            load_dst(dst_v, c1 + 1)
            pltpu.make_async_copy(x_hbm.at[src_b], rows_b, sem_b).wait()
            pltpu.sync_copy(rows_b, acc_sh.at[dst_b], add=True)
            return 0
        lax.fori_loop(0, STEPS // 2 - 1, pair, 0)

        load_src(src_b, STEPS - 1)
        pltpu.async_copy(x_hbm.at[src_b], rows_b, sem_b)
        load_dst(dst_b, STEPS - 1)
        pltpu.make_async_copy(x_hbm.at[src_v], rows_v, sem_a).wait()
        pltpu.sync_copy(rows_v, acc_sh.at[dst_v], add=True)
        pltpu.make_async_copy(x_hbm.at[src_b], rows_b, sem_b).wait()
        pltpu.sync_copy(rows_b, acc_sh.at[dst_b], add=True)

        plsc.subcore_barrier()
        out_r = c * HALF + base_r
        for j in range(RPS // K):
            pltpu.sync_copy(acc_sh.at[pl.ds(base_r + j * K, K)], rows_v)
            pltpu.sync_copy(rows_v, agg_out.at[pl.ds(out_r + j * K, K)])

    return agg_kernel(x, src, dst)


ES = 128
EP = 2560


def _tc_degree(dst2d):
    def body(dst_ref, o_ref):
        i = pl.program_id(0)

        @pl.when(i == 0)
        def _():
            o_ref[...] = jnp.zeros_like(o_ref)

        blk = dst_ref[...]
        h3 = blk // 128
        l3 = blk % 128
        rowh = lax.broadcasted_iota(jnp.int32, (NH, 128), 0)
        rowl = lax.broadcasted_iota(jnp.int32, (128, 128), 0)
        acc = jnp.zeros((NH, 128), jnp.float32)
        for r in range(ES):
            oh = (rowh == h3[r:r + 1, :]).astype(jnp.bfloat16)
            olt = (rowl == l3[r:r + 1, :]).astype(jnp.bfloat16)
            acc = acc + lax.dot_general(
                oh, olt, (((1,), (1,)), ((), ())),
                preferred_element_type=jnp.float32)
        o_ref[...] += acc

        @pl.when(i == pl.num_programs(0) - 1)
        def _():
            o_ref[...] = 1.0 / jnp.maximum(o_ref[...], 1.0)

    return pl.pallas_call(
        body,
        grid=(EP // ES,),
        in_specs=[pl.BlockSpec((ES, 128), lambda i: (i, 0))],
        out_specs=pl.BlockSpec((NH, 128), lambda i: (0, 0)),
        out_shape=jax.ShapeDtypeStruct((NH, 128), jnp.float32),
    )(dst2d)


R = 400


def _tc_combine(agg, dinv, x, w1, w2, b2):
    def body(agg_ref, dinv_ref, x_ref, w1_ref, w2_ref, b_ref, o_ref):
        dn = (((1,), (1,)), ((), ()))
        a = agg_ref[...] * dinv_ref[...]
        acc = lax.dot_general(a, w1_ref[...], dn,
                              preferred_element_type=jnp.float32)
        acc = acc + lax.dot_general(x_ref[...], w2_ref[...], dn,
                                    preferred_element_type=jnp.float32)
        o_ref[...] = acc + b_ref[...]

    return pl.pallas_call(
        body,
        grid=(N_NODES // R,),
        in_specs=[
            pl.BlockSpec((R, D), lambda i: (i, 0)),
            pl.BlockSpec((R, D), lambda i: (i, 0)),
            pl.BlockSpec((R, D), lambda i: (i, 0)),
            pl.BlockSpec((D, D), lambda i: (0, 0)),
            pl.BlockSpec((D, D), lambda i: (0, 0)),
            pl.BlockSpec((1, D), lambda i: (0, 0)),
        ],
        out_specs=pl.BlockSpec((R, D), lambda i: (i, 0)),
        out_shape=jax.ShapeDtypeStruct((N_NODES, D), jnp.float32),
    )(agg, dinv, x, w1, w2, b2)


def kernel(x, edge_index, W, b):
    src = edge_index[0].astype(jnp.int32)
    dst = edge_index[1].astype(jnp.int32)
    aggp = _sc_aggregate(x, src, dst)
    dst_pad = jnp.concatenate(
        [dst, jnp.full((EP * 128 - E,), 16383, jnp.int32)])
    dinv = _tc_degree(dst_pad.reshape(EP, 128))
    dinv_b = jnp.broadcast_to(
        dinv.reshape(NH * 128)[:N_NODES, None], (N_NODES, D))
    return _tc_combine(aggp[:N_NODES], dinv_b, x,
                       W[:, :D], W[:, D:], b.reshape(1, D))


# speedup vs baseline: 7.5249x; 1.0466x over previous
import functools

import jax
import jax.numpy as jnp
from jax import lax
from jax.experimental import pallas as pl
from jax.experimental.pallas import tpu as pltpu
from jax.experimental.pallas import tpu_sc as plsc

N_NODES = 10000
D = 128
E = 320000
NC, NS = 2, 16
HALF = 5120
ACC_R = HALF + 8
RPS = HALF // NS
E_W = E // NS
K = 80
STEPS = E_W // K
NH = (N_NODES + 127) // 128 + 1


def _sc_aggregate(x, src, dst):
    mesh = plsc.VectorSubcoreMesh(core_axis_name="c", subcore_axis_name="s")

    @functools.partial(
        pl.kernel,
        mesh=mesh,
        out_type=jax.ShapeDtypeStruct((NC * HALF, D), jnp.float32),
        scratch_types=[
            pltpu.VMEM((K,), jnp.int32),
            pltpu.VMEM((K, D), jnp.float32),
            pltpu.VMEM((K,), jnp.int32),
            pltpu.VMEM((K, D), jnp.float32),
            pltpu.VMEM((K,), jnp.int32),
            pltpu.VMEM((K, D), jnp.float32),
            pltpu.VMEM((K,), jnp.int32),
            pltpu.VMEM((K, D), jnp.float32),
            pltpu.VMEM((E_W,), jnp.int32),
            pltpu.VMEM((E_W,), jnp.int32),
            pltpu.VMEM_SHARED((ACC_R, D), jnp.float32),
            pltpu.SemaphoreType.DMA,
            pltpu.SemaphoreType.DMA,
            pltpu.SemaphoreType.DMA,
            pltpu.SemaphoreType.DMA,
        ],
    )
    def agg_kernel(x_hbm, src_hbm, dst_hbm, agg_out,
                   dst_v, rows_v, dst_b, rows_b, dst_c, rows_c, dst_d, rows_d,
                   srcall_v, dstall_v, acc_sh, sem_a, sem_b, sem_c, sem_d):
        c = lax.axis_index("c")
        s = lax.axis_index("s")

        zero16 = jnp.zeros((16,), jnp.float32)

        def zero_rows(i, _):
            r = i // (D // 16)
            j = i % (D // 16)
            rows_v[r, pl.ds(j * 16, 16)] = zero16
            return 0
        lax.fori_loop(0, K * (D // 16), zero_rows, 0)

        base_r = s * RPS
        for j in range(RPS // K):
            pltpu.sync_copy(rows_v, acc_sh.at[pl.ds(base_r + j * K, K)])
        pltpu.sync_copy(rows_v.at[pl.ds(0, 8)], acc_sh.at[pl.ds(HALF, 8)])

        plsc.subcore_barrier()

        lo = c * HALF

        pltpu.sync_copy(src_hbm.at[pl.ds(s * E_W, E_W)], srcall_v)
        pltpu.sync_copy(dst_hbm.at[pl.ds(s * E_W, E_W)], dstall_v)

        def rm_all(j, _):
            d16 = dstall_v[pl.ds(j * 16, 16)] - lo
            keep = (d16 >= 0) & (d16 < HALF)
            dstall_v[pl.ds(j * 16, 16)] = jnp.where(keep, d16, HALF)
            return 0
        lax.fori_loop(0, E_W // 16, rm_all, 0)

        def load_dst(buf, cidx):
            for t in range(K // 16):
                buf[pl.ds(t * 16, 16)] = dstall_v[pl.ds(cidx * K + t * 16, 16)]

        def gather_start(rows, sem, cidx):
            pltpu.async_copy(
                x_hbm.at[srcall_v.at[pl.ds(cidx * K, K)]], rows, sem)

        def gather_wait(rows, sem):
            pltpu.make_async_copy(x_hbm.at[pl.ds(0, K)], rows, sem).wait()

        slots = ((dst_v, rows_v, sem_a), (dst_b, rows_b, sem_b),
                 (dst_c, rows_c, sem_c), (dst_d, rows_d, sem_d))
        load_dst(dst_v, 0)
        gather_start(rows_v, sem_a, 0)
        load_dst(dst_b, 1)
        gather_start(rows_b, sem_b, 1)

        def quad(q, _):
            c0 = 4 * q
            for k in range(4):
                d_nxt, r_nxt, s_nxt = slots[(k + 2) % 4]
                d_cur, r_cur, s_cur = slots[k]
                load_dst(d_nxt, c0 + k + 2)
                gather_start(r_nxt, s_nxt, c0 + k + 2)
                gather_wait(r_cur, s_cur)
                pltpu.sync_copy(r_cur, acc_sh.at[d_cur], add=True)
            return 0
        lax.fori_loop(0, STEPS // 4, quad, 0)

        gather_wait(rows_v, sem_a)
        pltpu.sync_copy(rows_v, acc_sh.at[dst_v], add=True)
        gather_wait(rows_b, sem_b)
        pltpu.sync_copy(rows_b, acc_sh.at[dst_b], add=True)

        plsc.subcore_barrier()
        out_r = c * HALF + base_r
        for j in range(RPS // K):
            pltpu.sync_copy(acc_sh.at[pl.ds(base_r + j * K, K)], rows_v)
            pltpu.sync_copy(rows_v, agg_out.at[pl.ds(out_r + j * K, K)])

    return agg_kernel(x, src, dst)


ES = 128
EP = 2560


def _tc_degree(dst2d):
    def body(dst_ref, o_ref):
        i = pl.program_id(0)

        @pl.when(i == 0)
        def _():
            o_ref[...] = jnp.zeros_like(o_ref)

        blk = dst_ref[...]
        h3 = blk // 128
        l3 = blk % 128
        rowh = lax.broadcasted_iota(jnp.int32, (NH, 128), 0)
        rowl = lax.broadcasted_iota(jnp.int32, (128, 128), 0)
        acc = jnp.zeros((NH, 128), jnp.float32)
        for r in range(ES):
            oh = (rowh == h3[r:r + 1, :]).astype(jnp.bfloat16)
            olt = (rowl == l3[r:r + 1, :]).astype(jnp.bfloat16)
            acc = acc + lax.dot_general(
                oh, olt, (((1,), (1,)), ((), ())),
                preferred_element_type=jnp.float32)
        o_ref[...] += acc

        @pl.when(i == pl.num_programs(0) - 1)
        def _():
            o_ref[...] = 1.0 / jnp.maximum(o_ref[...], 1.0)

    return pl.pallas_call(
        body,
        grid=(EP // ES,),
        in_specs=[pl.BlockSpec((ES, 128), lambda i: (i, 0))],
        out_specs=pl.BlockSpec((NH, 128), lambda i: (0, 0)),
        out_shape=jax.ShapeDtypeStruct((NH, 128), jnp.float32),
    )(dst2d)


R = 400


def _tc_combine(agg, dinv, x, w1, w2, b2):
    def body(agg_ref, dinv_ref, x_ref, w1_ref, w2_ref, b_ref, o_ref):
        dn = (((1,), (1,)), ((), ()))
        a = agg_ref[...] * dinv_ref[...]
        acc = lax.dot_general(a, w1_ref[...], dn,
                              preferred_element_type=jnp.float32)
        acc = acc + lax.dot_general(x_ref[...], w2_ref[...], dn,
                                    preferred_element_type=jnp.float32)
        o_ref[...] = acc + b_ref[...]

    return pl.pallas_call(
        body,
        grid=(N_NODES // R,),
        in_specs=[
            pl.BlockSpec((R, D), lambda i: (i, 0)),
            pl.BlockSpec((R, 1), lambda i: (i, 0)),
            pl.BlockSpec((R, D), lambda i: (i, 0)),
            pl.BlockSpec((D, D), lambda i: (0, 0)),
            pl.BlockSpec((D, D), lambda i: (0, 0)),
            pl.BlockSpec((1, D), lambda i: (0, 0)),
        ],
        out_specs=pl.BlockSpec((R, D), lambda i: (i, 0)),
        out_shape=jax.ShapeDtypeStruct((N_NODES, D), jnp.float32),
    )(agg, dinv, x, w1, w2, b2)


def kernel(x, edge_index, W, b):
    src = edge_index[0].astype(jnp.int32)
    dst = edge_index[1].astype(jnp.int32)
    aggp = _sc_aggregate(x, src, dst)
    dst_pad = jnp.concatenate(
        [dst, jnp.full((EP * 128 - E,), 16383, jnp.int32)])
    dinv = _tc_degree(dst_pad.reshape(EP, 128))
    dinv_c = dinv.reshape(NH * 128, 1)
    return _tc_combine(aggp, dinv_c, x,
                       W[:, :D], W[:, D:], b.reshape(1, D))
